# baseline (speedup 1.0000x reference)
import functools

import jax
import jax.numpy as jnp
from jax import lax
from jax.experimental import pallas as pl
from jax.experimental.pallas import tpu as pltpu
from jax.experimental.pallas import tpu_sc as plsc

B, N, D = 4, 4096, 1024

NC, NS = 2, 16
NW = NC * NS
ROWS_PER_W = N // NW
CH = 32
NCHUNK = ROWS_PER_W // CH

_mesh = plsc.VectorSubcoreMesh(core_axis_name="c", subcore_axis_name="s")


@functools.partial(
    pl.kernel,
    out_type=jax.ShapeDtypeStruct((B, N, D), jnp.float32),
    mesh=_mesh,
    scratch_types=[
        pltpu.VMEM((CH, D), jnp.float32),
        pltpu.VMEM((CH, D), jnp.float32),
        pltpu.SemaphoreType.DMA,
        pltpu.SemaphoreType.DMA,
    ],
)
def _pos_embed_sc(table_hbm, out_hbm, buf0, buf1, sem_r, sem_w):
    wid = lax.axis_index("s") * NC + lax.axis_index("c")
    bufs = (buf0, buf1)

    def start_read(g):
        base = wid * ROWS_PER_W + g * CH
        return pltpu.async_copy(table_hbm.at[pl.ds(base, CH)], bufs[g % 2], sem_r)

    def fire_writes(g):
        base = wid * ROWS_PER_W + g * CH
        return [
            pltpu.async_copy(bufs[g % 2], out_hbm.at[b, pl.ds(base, CH)], sem_w)
            for b in range(B)
        ]

    reads = [None] * NCHUNK
    writes = [None] * NCHUNK
    reads[0] = start_read(0)
    for g in range(NCHUNK):
        reads[g].wait()
        writes[g] = fire_writes(g)
        if g + 1 < NCHUNK:
            if g >= 1:
                for c in writes[g - 1]:
                    c.wait()
            reads[g + 1] = start_read(g + 1)
    for g in (NCHUNK - 2, NCHUNK - 1):
        for c in writes[g]:
            c.wait()


def kernel(patches, table):
    del patches
    return _pos_embed_sc(table)
